# baseline (speedup 1.0000x reference)
import functools

import jax
import jax.numpy as jnp
from jax import lax
from jax.experimental import pallas as pl
from jax.experimental.pallas import tpu as pltpu
from jax.experimental.pallas import tpu_sc as plsc

N = 100000
D = 128
B = 256

NC = 2
NS = 16
NW = NC * NS

G = 100

S1 = 400
NCH1 = N // S1
IT1 = -(-NCH1 // NW)
J1 = S1 // G

S3 = 160
NCH3 = N // S3
IT3 = -(-NCH3 // NW)

_mesh = plsc.VectorSubcoreMesh(core_axis_name="c", subcore_axis_name="s")


def _issue_loads(h_hbm, bat_hbm, cid, rows, hb, ibs, sem):
    for j in range(len(ibs)):
        pltpu.async_copy(bat_hbm.at[cid, j], ibs[j], sem)
    pltpu.async_copy(h_hbm.at[pl.ds(cid * rows, rows), :], hb, sem)


def _wait_loads(h_hbm, bat_hbm, cid, rows, hb, ibs, sem):
    for j in range(len(ibs)):
        pltpu.make_async_copy(bat_hbm.at[cid, j], ibs[j], sem).wait()
    pltpu.make_async_copy(h_hbm.at[pl.ds(cid * rows, rows), :], hb, sem).wait()


@functools.partial(
    pl.kernel,
    out_type=jax.ShapeDtypeStruct((NC, B, D), jnp.float32),
    mesh=_mesh,
    scratch_types=[
        (pltpu.VMEM((G,), jnp.int32),) * J1,
        (pltpu.VMEM((G,), jnp.int32),) * J1,
        pltpu.VMEM((S1, D), jnp.float32),
        pltpu.VMEM((S1, D), jnp.float32),
        pltpu.VMEM_SHARED((B, D), jnp.float32),
        pltpu.SemaphoreType.DMA,
        pltpu.SemaphoreType.DMA,
    ],
)
def _sc_pool(h_hbm, bat_hbm, zero_hbm, out_hbm,
             ib0, ib1, hb0, hb1, acc, semL0, semL1):
    c = lax.axis_index("c")
    s = lax.axis_index("s")
    wid = s * NC + c

    @pl.when(s == 0)
    def _init():
        pltpu.sync_copy(zero_hbm, acc)

    plsc.subcore_barrier()

    _issue_loads(h_hbm, bat_hbm, wid, S1, hb0, ib0, semL0)

    def _scatter(hb, ibs):
        for j in range(J1):
            pltpu.sync_copy(hb.at[pl.ds(j * G, G), :], acc.at[ibs[j]],
                            add=True)

    def body(k, carry):
        c0 = (2 * k) * NW + wid
        c1 = c0 + NW
        c2 = c0 + 2 * NW

        @pl.when(c0 < NCH1)
        def _slot0():
            _wait_loads(h_hbm, bat_hbm, c0, S1, hb0, ib0, semL0)

            @pl.when(c1 < NCH1)
            def _():
                _issue_loads(h_hbm, bat_hbm, c1, S1, hb1, ib1, semL1)

            _scatter(hb0, ib0)

        @pl.when(c1 < NCH1)
        def _slot1():
            _wait_loads(h_hbm, bat_hbm, c1, S1, hb1, ib1, semL1)

            @pl.when(c2 < NCH1)
            def _():
                _issue_loads(h_hbm, bat_hbm, c2, S1, hb0, ib0, semL0)

            _scatter(hb1, ib1)

        return carry

    lax.fori_loop(0, -(-IT1 // 2), body, 0)
    plsc.subcore_barrier()

    @pl.when(s == 0)
    def _flush():
        pltpu.sync_copy(acc, out_hbm.at[c])


NB = 3
KIT3 = -(-IT3 // NB)


SD3 = S3 * D


def _issue_loads3(h_hbm, bat_hbm, cid, hb, ib, sem):
    pltpu.async_copy(bat_hbm.at[cid], ib, sem)
    pltpu.async_copy(h_hbm.at[pl.ds(cid * SD3, SD3)], hb, sem)


def _wait_loads3(h_hbm, bat_hbm, cid, hb, ib, sem):
    pltpu.make_async_copy(bat_hbm.at[cid], ib, sem).wait()
    pltpu.make_async_copy(h_hbm.at[pl.ds(cid * SD3, SD3)], hb, sem).wait()


@functools.partial(
    pl.kernel,
    out_type=jax.ShapeDtypeStruct((N * D,), jnp.float32),
    mesh=_mesh,
    scratch_types=[
        (pltpu.VMEM((S3,), jnp.int32),) * NB,
        (pltpu.VMEM((SD3,), jnp.float32),) * NB,
        pltpu.VMEM((B * D,), jnp.float32),
        (pltpu.SemaphoreType.DMA,) * NB,
        (pltpu.SemaphoreType.DMA,) * NB,
        pltpu.SemaphoreType.DMA,
    ],
    compiler_params=pltpu.CompilerParams(needs_layout_passes=False),
)
def _sc_broadcast(h_hbm, bat_hbm, vn_hbm, out_hbm,
                  ibs, hbs, vnb, semL, semO, semV):
    c = lax.axis_index("c")
    s = lax.axis_index("s")
    wid = s * NC + c

    pltpu.async_copy(vn_hbm, vnb, semV)
    _issue_loads3(h_hbm, bat_hbm, wid, hbs[0], ibs[0], semL[0])
    _issue_loads3(h_hbm, bat_hbm, wid + NW, hbs[1], ibs[1], semL[1])
    pltpu.make_async_copy(vn_hbm, vnb, semV).wait()

    lanes = lax.iota(jnp.int32, 16)
    lanesD = lanes * D

    def _compute(p):
        hb, ib = hbs[p], ibs[p]

        def grp(g, cg):
            bv = ib[pl.ds(g * 16, 16)]
            av = bv * D
            rv = lanesD + g * (16 * D)

            def colblk(cb, cc_):
                c0 = cb * 8
                for cc in range(8):
                    hv = plsc.load_gather(hb, [rv + (c0 + cc)])
                    vv = plsc.load_gather(vnb, [av + (c0 + cc)])
                    plsc.store_scatter(hb, [rv + (c0 + cc)], hv + vv)
                return cc_

            lax.fori_loop(0, D // 8, colblk, 0)
            return cg

        lax.fori_loop(0, S3 // 16, grp, 0)

    def section(k, p, first):
        cid = (NB * k + p) * NW + wid
        c2 = cid + 2 * NW
        p2 = (p + 2) % NB

        @pl.when(cid < NCH3)
        def _():
            _wait_loads3(h_hbm, bat_hbm, cid, hbs[p], ibs[p], semL[p])

            @pl.when(c2 < NCH3)
            def _():
                if not first:
                    pltpu.make_async_copy(
                        hbs[p2], out_hbm.at[pl.ds(0, SD3)], semO[p2]).wait()
                _issue_loads3(h_hbm, bat_hbm, c2, hbs[p2], ibs[p2],
                              semL[p2])

            _compute(p)
            pltpu.async_copy(hbs[p], out_hbm.at[pl.ds(cid * SD3, SD3)],
                             semO[p])

    section(0, 0, True)
    section(0, 1, False)
    section(0, 2, False)

    def body(k, carry):
        for p in range(NB):
            section(k, p, False)
        return carry

    lax.fori_loop(1, KIT3, body, 0)

    for p in range(NB):
        pltpu.make_async_copy(hbs[p], out_hbm.at[pl.ds(0, SD3)],
                              semO[p]).wait()


def _fc_body(p_ref, v_ref, w_ref, b_ref, o_ref):
    vn = v_ref[...]
    z = vn + p_ref[0] + p_ref[1]
    y = jnp.dot(z, w_ref[...], preferred_element_type=jnp.float32) + b_ref[...]
    o_ref[...] = vn + jnp.maximum(y, 0.0)


_fc = pl.pallas_call(
    _fc_body,
    out_shape=jax.ShapeDtypeStruct((B, D), jnp.float32),
)


def kernel(h, vn_h, batch, W, b):
    batch_i = batch.astype(jnp.int32)
    bat1 = batch_i.reshape(NCH1, J1, G)
    bat3 = batch_i.reshape(NCH3, S3)
    zero = jnp.zeros((B, D), jnp.float32)
    pool2 = _sc_pool(h, bat1, zero)
    vn_new = _fc(pool2, vn_h, W, b.reshape(1, D))
    h_out = _sc_broadcast(h.reshape(N * D), bat3, vn_new.reshape(B * D))
    return h_out.reshape(N, D), vn_new


# speedup vs baseline: 1.5046x; 1.5046x over previous
import functools

import jax
import jax.numpy as jnp
from jax import lax
from jax.experimental import pallas as pl
from jax.experimental.pallas import tpu as pltpu
from jax.experimental.pallas import tpu_sc as plsc

N = 100000
D = 128
B = 256

NC = 2
NS = 16
NW = NC * NS

G = 100

S1 = 400
NCH1 = N // S1
IT1 = -(-NCH1 // NW)
J1 = S1 // G

S3 = 160
NCH3 = N // S3
IT3 = -(-NCH3 // NW)

_mesh = plsc.VectorSubcoreMesh(core_axis_name="c", subcore_axis_name="s")


def _issue_loads(h_hbm, bat_hbm, cid, rows, hb, ibs, sem):
    for j in range(len(ibs)):
        pltpu.async_copy(bat_hbm.at[cid, j], ibs[j], sem)
    pltpu.async_copy(h_hbm.at[pl.ds(cid * rows, rows), :], hb, sem)


def _wait_loads(h_hbm, bat_hbm, cid, rows, hb, ibs, sem):
    for j in range(len(ibs)):
        pltpu.make_async_copy(bat_hbm.at[cid, j], ibs[j], sem).wait()
    pltpu.make_async_copy(h_hbm.at[pl.ds(cid * rows, rows), :], hb, sem).wait()


@functools.partial(
    pl.kernel,
    out_type=jax.ShapeDtypeStruct((NC, B, D), jnp.float32),
    mesh=_mesh,
    scratch_types=[
        (pltpu.VMEM((G,), jnp.int32),) * J1,
        (pltpu.VMEM((G,), jnp.int32),) * J1,
        pltpu.VMEM((S1, D), jnp.float32),
        pltpu.VMEM((S1, D), jnp.float32),
        pltpu.VMEM_SHARED((B, D), jnp.float32),
        pltpu.SemaphoreType.DMA,
        pltpu.SemaphoreType.DMA,
    ],
)
def _sc_pool(h_hbm, bat_hbm, zero_hbm, out_hbm,
             ib0, ib1, hb0, hb1, acc, semL0, semL1):
    c = lax.axis_index("c")
    s = lax.axis_index("s")
    wid = s * NC + c

    @pl.when(s == 0)
    def _init():
        pltpu.sync_copy(zero_hbm, acc)

    plsc.subcore_barrier()

    _issue_loads(h_hbm, bat_hbm, wid, S1, hb0, ib0, semL0)

    def _scatter(hb, ibs):
        for j in range(J1):
            pltpu.sync_copy(hb.at[pl.ds(j * G, G), :], acc.at[ibs[j]],
                            add=True)

    def body(k, carry):
        c0 = (2 * k) * NW + wid
        c1 = c0 + NW
        c2 = c0 + 2 * NW

        @pl.when(c0 < NCH1)
        def _slot0():
            _wait_loads(h_hbm, bat_hbm, c0, S1, hb0, ib0, semL0)

            @pl.when(c1 < NCH1)
            def _():
                _issue_loads(h_hbm, bat_hbm, c1, S1, hb1, ib1, semL1)

            _scatter(hb0, ib0)

        @pl.when(c1 < NCH1)
        def _slot1():
            _wait_loads(h_hbm, bat_hbm, c1, S1, hb1, ib1, semL1)

            @pl.when(c2 < NCH1)
            def _():
                _issue_loads(h_hbm, bat_hbm, c2, S1, hb0, ib0, semL0)

            _scatter(hb1, ib1)

        return carry

    lax.fori_loop(0, -(-IT1 // 2), body, 0)
    plsc.subcore_barrier()

    @pl.when(s == 0)
    def _flush():
        pltpu.sync_copy(acc, out_hbm.at[c])


NB = 3
KIT3 = -(-IT3 // NB)


SD3 = S3 * D


def _issue_loads3(h_hbm, bat_hbm, cid, hb, ib, sem):
    pltpu.async_copy(bat_hbm.at[cid], ib, sem)
    pltpu.async_copy(h_hbm.at[pl.ds(cid * SD3, SD3)], hb, sem)


def _wait_loads3(h_hbm, bat_hbm, cid, hb, ib, sem):
    pltpu.make_async_copy(bat_hbm.at[cid], ib, sem).wait()
    pltpu.make_async_copy(h_hbm.at[pl.ds(cid * SD3, SD3)], hb, sem).wait()


@functools.partial(
    pl.kernel,
    out_type=jax.ShapeDtypeStruct((N * D,), jnp.float32),
    mesh=_mesh,
    scratch_types=[
        (pltpu.VMEM((S3,), jnp.int32),) * NB,
        (pltpu.VMEM((SD3,), jnp.float32),) * NB,
        pltpu.VMEM((S3, D), jnp.float32),
        (pltpu.SemaphoreType.DMA,) * NB,
        (pltpu.SemaphoreType.DMA,) * NB,
        pltpu.SemaphoreType.DMA,
    ],
)
def _sc_broadcast(h_hbm, bat_hbm, vn_hbm, out_hbm,
                  ibs, hbs, vb, semL, semO, semG):
    c = lax.axis_index("c")
    s = lax.axis_index("s")
    wid = s * NC + c

    _issue_loads3(h_hbm, bat_hbm, wid, hbs[0], ibs[0], semL[0])
    _issue_loads3(h_hbm, bat_hbm, wid + NW, hbs[1], ibs[1], semL[1])

    GH = S3 // 2

    def _compute(p):
        hb, ib = hbs[p], ibs[p]
        for j in range(2):
            pltpu.async_copy(vn_hbm.at[ib.at[pl.ds(j * GH, GH)]],
                             vb.at[pl.ds(j * GH, GH), :], semG)
        for j in range(2):
            pltpu.make_async_copy(vn_hbm.at[ib.at[pl.ds(j * GH, GH)]],
                                  vb.at[pl.ds(j * GH, GH), :], semG).wait()

        def row(r, cg):
            base = r * D
            for k in range(D // 16):
                sl = pl.ds(base + k * 16, 16)
                hb[sl] = hb[sl] + vb[r, pl.ds(k * 16, 16)]
            return cg

        lax.fori_loop(0, S3, row, 0)

    def section(k, p, first):
        cid = (NB * k + p) * NW + wid
        c2 = cid + 2 * NW
        p2 = (p + 2) % NB

        @pl.when(cid < NCH3)
        def _():
            _wait_loads3(h_hbm, bat_hbm, cid, hbs[p], ibs[p], semL[p])

            @pl.when(c2 < NCH3)
            def _():
                if not first:
                    pltpu.make_async_copy(
                        hbs[p2], out_hbm.at[pl.ds(0, SD3)], semO[p2]).wait()
                _issue_loads3(h_hbm, bat_hbm, c2, hbs[p2], ibs[p2],
                              semL[p2])

            _compute(p)
            pltpu.async_copy(hbs[p], out_hbm.at[pl.ds(cid * SD3, SD3)],
                             semO[p])

    section(0, 0, True)
    section(0, 1, False)
    section(0, 2, False)

    def body(k, carry):
        for p in range(NB):
            section(k, p, False)
        return carry

    lax.fori_loop(1, KIT3, body, 0)

    for p in range(NB):
        pltpu.make_async_copy(hbs[p], out_hbm.at[pl.ds(0, SD3)],
                              semO[p]).wait()


def _fc_body(p_ref, v_ref, w_ref, b_ref, o_ref):
    vn = v_ref[...]
    z = vn + p_ref[0] + p_ref[1]
    y = jnp.dot(z, w_ref[...], preferred_element_type=jnp.float32) + b_ref[...]
    o_ref[...] = vn + jnp.maximum(y, 0.0)


_fc = pl.pallas_call(
    _fc_body,
    out_shape=jax.ShapeDtypeStruct((B, D), jnp.float32),
)


def kernel(h, vn_h, batch, W, b):
    batch_i = batch.astype(jnp.int32)
    bat1 = batch_i.reshape(NCH1, J1, G)
    bat3 = batch_i.reshape(NCH3, S3)
    zero = jnp.zeros((B, D), jnp.float32)
    pool2 = _sc_pool(h, bat1, zero)
    vn_new = _fc(pool2, vn_h, W, b.reshape(1, D))
    h_out = _sc_broadcast(h.reshape(N * D), bat3, vn_new)
    return h_out.reshape(N, D), vn_new


# speedup vs baseline: 2.7401x; 1.8212x over previous
import functools

import jax
import jax.numpy as jnp
from jax import lax
from jax.experimental import pallas as pl
from jax.experimental.pallas import tpu as pltpu
from jax.experimental.pallas import tpu_sc as plsc

N = 100000
D = 128
B = 256

NC = 2
NS = 16
NW = NC * NS

G = 100

S1 = 400
NCH1 = N // S1
IT1 = -(-NCH1 // NW)
J1 = S1 // G

S3 = 160
NCH3 = N // S3
IT3 = -(-NCH3 // NW)

_mesh = plsc.VectorSubcoreMesh(core_axis_name="c", subcore_axis_name="s")


def _issue_loads(h_hbm, bat_hbm, cid, rows, hb, ibs, sem):
    for j in range(len(ibs)):
        pltpu.async_copy(bat_hbm.at[cid, j], ibs[j], sem)
    pltpu.async_copy(h_hbm.at[pl.ds(cid * rows, rows), :], hb, sem)


def _wait_loads(h_hbm, bat_hbm, cid, rows, hb, ibs, sem):
    for j in range(len(ibs)):
        pltpu.make_async_copy(bat_hbm.at[cid, j], ibs[j], sem).wait()
    pltpu.make_async_copy(h_hbm.at[pl.ds(cid * rows, rows), :], hb, sem).wait()


@functools.partial(
    pl.kernel,
    out_type=jax.ShapeDtypeStruct((NC, B, D), jnp.float32),
    mesh=_mesh,
    scratch_types=[
        (pltpu.VMEM((G,), jnp.int32),) * J1,
        (pltpu.VMEM((G,), jnp.int32),) * J1,
        pltpu.VMEM((S1, D), jnp.float32),
        pltpu.VMEM((S1, D), jnp.float32),
        pltpu.VMEM_SHARED((B, D), jnp.float32),
        pltpu.SemaphoreType.DMA,
        pltpu.SemaphoreType.DMA,
    ],
)
def _sc_pool(h_hbm, bat_hbm, zero_hbm, out_hbm,
             ib0, ib1, hb0, hb1, acc, semL0, semL1):
    c = lax.axis_index("c")
    s = lax.axis_index("s")
    wid = s * NC + c

    @pl.when(s == 0)
    def _init():
        pltpu.sync_copy(zero_hbm, acc)

    plsc.subcore_barrier()

    _issue_loads(h_hbm, bat_hbm, wid, S1, hb0, ib0, semL0)

    def _scatter(hb, ibs):
        for j in range(J1):
            pltpu.sync_copy(hb.at[pl.ds(j * G, G), :], acc.at[ibs[j]],
                            add=True)

    def body(k, carry):
        c0 = (2 * k) * NW + wid
        c1 = c0 + NW
        c2 = c0 + 2 * NW

        @pl.when(c0 < NCH1)
        def _slot0():
            _wait_loads(h_hbm, bat_hbm, c0, S1, hb0, ib0, semL0)

            @pl.when(c1 < NCH1)
            def _():
                _issue_loads(h_hbm, bat_hbm, c1, S1, hb1, ib1, semL1)

            _scatter(hb0, ib0)

        @pl.when(c1 < NCH1)
        def _slot1():
            _wait_loads(h_hbm, bat_hbm, c1, S1, hb1, ib1, semL1)

            @pl.when(c2 < NCH1)
            def _():
                _issue_loads(h_hbm, bat_hbm, c2, S1, hb0, ib0, semL0)

            _scatter(hb1, ib1)

        return carry

    lax.fori_loop(0, -(-IT1 // 2), body, 0)
    plsc.subcore_barrier()

    @pl.when(s == 0)
    def _flush():
        pltpu.sync_copy(acc, out_hbm.at[c])


NB = 3
KIT3 = -(-IT3 // NB)


SD3 = S3 * D


def _issue_loads3(h_hbm, bat_hbm, cid, hb, ib, sem):
    pltpu.async_copy(bat_hbm.at[cid], ib, sem)
    pltpu.async_copy(h_hbm.at[pl.ds(cid * SD3, SD3)], hb, sem)


def _wait_loads3(h_hbm, bat_hbm, cid, hb, ib, sem):
    pltpu.make_async_copy(bat_hbm.at[cid], ib, sem).wait()
    pltpu.make_async_copy(h_hbm.at[pl.ds(cid * SD3, SD3)], hb, sem).wait()


@functools.partial(
    pl.kernel,
    out_type=jax.ShapeDtypeStruct((N * D,), jnp.float32),
    mesh=_mesh,
    scratch_types=[
        (pltpu.VMEM((S3,), jnp.int32),) * NB,
        (pltpu.VMEM((SD3,), jnp.float32),) * NB,
        pltpu.VMEM((B * D,), jnp.float32),
        (pltpu.SemaphoreType.DMA,) * NB,
        (pltpu.SemaphoreType.DMA,) * NB,
        pltpu.SemaphoreType.DMA,
    ],
    compiler_params=pltpu.CompilerParams(needs_layout_passes=False),
)
def _sc_broadcast(h_hbm, bat_hbm, vn_hbm, out_hbm,
                  ibs, hbs, vnb, semL, semO, semV):
    c = lax.axis_index("c")
    s = lax.axis_index("s")
    wid = s * NC + c

    pltpu.async_copy(vn_hbm, vnb, semV)
    _issue_loads3(h_hbm, bat_hbm, wid, hbs[0], ibs[0], semL[0])
    _issue_loads3(h_hbm, bat_hbm, wid + NW, hbs[1], ibs[1], semL[1])
    pltpu.make_async_copy(vn_hbm, vnb, semV).wait()

    lanes = lax.iota(jnp.int32, 16)

    def _compute(p):
        hb, ib = hbs[p], ibs[p]

        def grp(g, cg):
            bv = ib[pl.ds(g * 16, 16)]
            gmax = jnp.max(bv)
            gmin = jnp.min(bv)
            base = g * 16 * D

            def uniform():
                voff = gmax * D
                vre = [vnb[pl.ds(voff + 16 * k, 16)] for k in range(8)]

                def rowu(r2, cg2):
                    o = base + r2 * D
                    for k in range(D // 16):
                        sl = pl.ds(o + k * 16, 16)
                        hb[sl] = hb[sl] + vre[k]
                    return cg2

                lax.fori_loop(0, 16, rowu, 0)

            def mixed():
                def rowm(r2, cg2):
                    voff = jnp.max(jnp.where(lanes == r2, bv, 0)) * D
                    o = base + r2 * D
                    for k in range(D // 16):
                        sl = pl.ds(o + k * 16, 16)
                        hb[sl] = hb[sl] + vnb[pl.ds(voff + k * 16, 16)]
                    return cg2

                lax.fori_loop(0, 16, rowm, 0)

            lax.cond(gmax == gmin, uniform, mixed)
            return cg

        lax.fori_loop(0, S3 // 16, grp, 0)

    def section(k, p, first):
        cid = (NB * k + p) * NW + wid
        c2 = cid + 2 * NW
        p2 = (p + 2) % NB

        @pl.when(cid < NCH3)
        def _():
            _wait_loads3(h_hbm, bat_hbm, cid, hbs[p], ibs[p], semL[p])

            @pl.when(c2 < NCH3)
            def _():
                if not first:
                    pltpu.make_async_copy(
                        hbs[p2], out_hbm.at[pl.ds(0, SD3)], semO[p2]).wait()
                _issue_loads3(h_hbm, bat_hbm, c2, hbs[p2], ibs[p2],
                              semL[p2])

            _compute(p)
            pltpu.async_copy(hbs[p], out_hbm.at[pl.ds(cid * SD3, SD3)],
                             semO[p])

    section(0, 0, True)
    section(0, 1, False)
    section(0, 2, False)

    def body(k, carry):
        for p in range(NB):
            section(k, p, False)
        return carry

    lax.fori_loop(1, KIT3, body, 0)

    for p in range(NB):
        pltpu.make_async_copy(hbs[p], out_hbm.at[pl.ds(0, SD3)],
                              semO[p]).wait()


def _fc_body(p_ref, v_ref, w_ref, b_ref, o_ref):
    vn = v_ref[...]
    z = vn + p_ref[0] + p_ref[1]
    y = jnp.dot(z, w_ref[...], preferred_element_type=jnp.float32) + b_ref[...]
    o_ref[...] = vn + jnp.maximum(y, 0.0)


_fc = pl.pallas_call(
    _fc_body,
    out_shape=jax.ShapeDtypeStruct((B, D), jnp.float32),
)


def kernel(h, vn_h, batch, W, b):
    batch_i = batch.astype(jnp.int32)
    bat1 = batch_i.reshape(NCH1, J1, G)
    bat3 = batch_i.reshape(NCH3, S3)
    zero = jnp.zeros((B, D), jnp.float32)
    pool2 = _sc_pool(h, bat1, zero)
    vn_new = _fc(pool2, vn_h, W, b.reshape(1, D))
    h_out = _sc_broadcast(h.reshape(N * D), bat3, vn_new.reshape(B * D))
    return h_out.reshape(N, D), vn_new
